# Initial kernel scaffold; baseline (speedup 1.0000x reference)
import jax
import jax.numpy as jnp
from jax.experimental import pallas as pl


def kernel(x, thetas, phis):
    raise NotImplementedError("write your pallas kernel here")


# speedup vs baseline: 1203.6948x; 1203.6948x over previous
import functools

import jax
import jax.numpy as jnp
from jax.experimental import pallas as pl
from jax.experimental.pallas import tpu as pltpu


def _build_u_kernel(th_ref, ph_ref, ur_ref, ui_ref,
                    uer, uei, uor, uoi,
                    car, cai, cbr, cbi, ccr, cci, cdr, cdi):
    n, m = uer.shape
    f32 = jnp.float32

    jj = jax.lax.broadcasted_iota(jnp.int32, (n, m), 0)
    kk = jax.lax.broadcasted_iota(jnp.int32, (n, m), 1)
    uer[...] = (jj == 2 * kk).astype(f32)
    uei[...] = jnp.zeros((n, m), f32)
    uor[...] = (jj == 2 * kk + 1).astype(f32)
    uoi[...] = jnp.zeros((n, m), f32)

    ct = jnp.cos(th_ref[...])
    st = jnp.sin(th_ref[...])
    cp = jnp.cos(ph_ref[...])
    sp = jnp.sin(ph_ref[...])
    ar = 0.5 * ((ct - 1.0) * cp - st * sp)
    ai = 0.5 * ((ct - 1.0) * sp + st * cp)
    br = -0.5 * st
    bi = 0.5 * (ct + 1.0)
    cr = br * cp - bi * sp
    ci = br * sp + bi * cp
    dr = 0.5 * (1.0 - ct)
    di = -0.5 * st
    crow = jax.lax.broadcasted_iota(jnp.int32, (n, m), 0)
    lane = jax.lax.broadcasted_iota(jnp.int32, (n, m), 1)
    edge = (crow % 2 == 1) & (lane == m - 1)
    zero = jnp.zeros((n, m), f32)
    one = jnp.ones((n, m), f32)
    car[...] = jnp.where(edge, one, ar).reshape(n, 1, m)
    cai[...] = jnp.where(edge, zero, ai).reshape(n, 1, m)
    cbr[...] = jnp.where(edge, zero, br).reshape(n, 1, m)
    cbi[...] = jnp.where(edge, zero, bi).reshape(n, 1, m)
    ccr[...] = jnp.where(edge, zero, cr).reshape(n, 1, m)
    cci[...] = jnp.where(edge, zero, ci).reshape(n, 1, m)
    cdr[...] = jnp.where(edge, one, dr).reshape(n, 1, m)
    cdi[...] = jnp.where(edge, zero, di).reshape(n, 1, m)

    def mix(c, t_r, t_i, w_r, w_i):
        a_r = car[pl.ds(c, 1)].reshape(1, m)
        a_i = cai[pl.ds(c, 1)].reshape(1, m)
        b_r = cbr[pl.ds(c, 1)].reshape(1, m)
        b_i = cbi[pl.ds(c, 1)].reshape(1, m)
        c_r = ccr[pl.ds(c, 1)].reshape(1, m)
        c_i = cci[pl.ds(c, 1)].reshape(1, m)
        d_r = cdr[pl.ds(c, 1)].reshape(1, m)
        d_i = cdi[pl.ds(c, 1)].reshape(1, m)
        nt_r = a_r * t_r - a_i * t_i + b_r * w_r - b_i * w_i
        nt_i = a_r * t_i + a_i * t_r + b_r * w_i + b_i * w_r
        nb_r = c_r * t_r - c_i * t_i + d_r * w_r - d_i * w_i
        nb_i = c_r * t_i + c_i * t_r + d_r * w_i + d_i * w_r
        return nt_r, nt_i, nb_r, nb_i

    def body(k, carry):
        nt_r, nt_i, nb_r, nb_i = mix(2 * k, uer[...], uei[...], uor[...], uoi[...])
        uer[...] = nt_r
        uei[...] = nt_i
        w_r = pltpu.roll(nt_r, m - 1, 1)
        w_i = pltpu.roll(nt_i, m - 1, 1)
        ot_r, ot_i, ob_r, ob_i = mix(2 * k + 1, nb_r, nb_i, w_r, w_i)
        uor[...] = ot_r
        uoi[...] = ot_i
        uer[...] = pltpu.roll(ob_r, 1, 1)
        uei[...] = pltpu.roll(ob_i, 1, 1)
        return carry

    jax.lax.fori_loop(0, m, body, 0)

    rr = jax.lax.broadcasted_iota(jnp.int32, (n, n), 0)
    qq = jax.lax.broadcasted_iota(jnp.int32, (n, n), 1)
    perm = (qq == (rr // 2 + m * (rr % 2))).astype(f32)
    s_r = jnp.concatenate([uer[...], uor[...]], axis=1)
    s_i = jnp.concatenate([uei[...], uoi[...]], axis=1)
    dn = (((1,), (1,)), ((), ()))
    ur_ref[...] = jax.lax.dot_general(perm, s_r, dn, preferred_element_type=f32)
    ui_ref[...] = jax.lax.dot_general(perm, s_i, dn, preferred_element_type=f32)


def _apply_kernel(x_ref, ur_ref, ui_ref, out_ref):
    yr = jnp.dot(ur_ref[...], x_ref[...], preferred_element_type=jnp.float32)
    yi = jnp.dot(ui_ref[...], x_ref[...], preferred_element_type=jnp.float32)
    out_ref[...] = yr * yr + yi * yi


@functools.partial(jax.jit, static_argnames=("interpret",))
def kernel(x, thetas, phis, interpret=False):
    n, b = x.shape
    m = n // 2
    f32 = jnp.float32

    ur, ui = pl.pallas_call(
        _build_u_kernel,
        out_shape=[jax.ShapeDtypeStruct((n, n), f32)] * 2,
        scratch_shapes=[pltpu.VMEM((n, m), f32)] * 4
        + [pltpu.VMEM((n, 1, m), f32)] * 8,
        interpret=interpret,
    )(thetas, phis)

    bblk = 512
    out = pl.pallas_call(
        _apply_kernel,
        grid=(b // bblk,),
        in_specs=[
            pl.BlockSpec((n, bblk), lambda i: (0, i)),
            pl.BlockSpec((n, n), lambda i: (0, 0)),
            pl.BlockSpec((n, n), lambda i: (0, 0)),
        ],
        out_specs=pl.BlockSpec((n, bblk), lambda i: (0, i)),
        out_shape=jax.ShapeDtypeStruct((n, b), f32),
        compiler_params=pltpu.CompilerParams(
            dimension_semantics=("arbitrary",),
        ),
        interpret=interpret,
    )(x, ur, ui)
    return out
